# baseline (speedup 1.0000x reference)
import functools

import jax
import jax.numpy as jnp
from jax import lax
from jax.experimental import pallas as pl
from jax.experimental.pallas import tpu as pltpu
from jax.experimental.pallas import tpu_sc as plsc

_V = 1000000
_D = 64
_B = 16384
_NEG = 5

_NC = 2
_NS = 16
_NW = _NC * _NS
_L = 16

_CHUNK = _B // _NW
_CH = 8
_NPAIR = _CHUNK // _L


def _sc_scores(u4, v4, pos_u, pos_v, negf):
    mesh = plsc.VectorSubcoreMesh(
        core_axis_name="c", subcore_axis_name="s",
        num_cores=_NC, num_subcores=_NS)

    @functools.partial(
        pl.kernel,
        out_type=(
            jax.ShapeDtypeStruct((_B,), jnp.float32),
            jax.ShapeDtypeStruct((_NEG * _B,), jnp.float32),
        ),
        mesh=mesh,
        scratch_types=[
            pltpu.VMEM((_CHUNK,), jnp.int32),
            pltpu.VMEM((_CHUNK,), jnp.int32),
            pltpu.VMEM((_CHUNK * _NEG,), jnp.int32),
            [[pltpu.VMEM((_CH, 8, _D), jnp.float32),
              pltpu.VMEM((_CH, 8, _D), jnp.float32),
              pltpu.VMEM((_CH * _NEG, 8, _D), jnp.float32)]
             for _ in range(2)],
            pltpu.VMEM((_CHUNK,), jnp.float32),
            [pltpu.VMEM((_CHUNK,), jnp.float32) for _ in range(_NEG)],
            [pltpu.SemaphoreType.DMA for _ in range(2)],
        ],
        compiler_params=pltpu.CompilerParams(needs_layout_passes=False),
    )
    def k(u_hbm, v_hbm, pu_hbm, pv_hbm, nv_hbm, pos_out, neg_out,
          idxu, idxv, idxn, bufs, psc, nsc, sems):
        wid = lax.axis_index("s") * _NC + lax.axis_index("c")
        base = wid * _CHUNK
        pltpu.sync_copy(pu_hbm.at[pl.ds(base, _CHUNK)], idxu)
        pltpu.sync_copy(pv_hbm.at[pl.ds(base, _CHUNK)], idxv)
        pltpu.sync_copy(nv_hbm.at[pl.ds(base * _NEG, _CHUNK * _NEG)], idxn)
        iota = lax.iota(jnp.int32, _L)
        lo_mask = iota < _CH

        def pair_vecs(g):
            iu = idxu[pl.ds(g * _L, _L)]
            iv = idxv[pl.ds(g * _L, _L)]
            inj = [idxn[pl.ds(g * _L * _NEG + _L * j, _L)]
                   for j in range(_NEG)]
            return iu, iv, inj

        def fire(vecs, b):
            iu, iv, inj = vecs
            tv = lax.shift_right_logical(iv, 4)
            hv = jnp.bitwise_and(lax.shift_right_logical(iv, 3), 1)
            tn = [lax.shift_right_logical(x, 4) for x in inj]
            hn = [jnp.bitwise_and(lax.shift_right_logical(x, 3), 1)
                  for x in inj]
            bu = jnp.bitwise_and(iu, -8)
            ub, vb, nb = bufs[b]
            for t in range(_CH):
                ll = _CH * b + t
                pltpu.async_copy(
                    u_hbm.at[pl.ds(pl.multiple_of(bu[ll], 8), 8)],
                    ub.at[t], sems[b])
                pltpu.async_copy(v_hbm.at[tv[ll], hv[ll]], vb.at[t], sems[b])
            for r in range(_CH * _NEG):
                p = _CH * _NEG * b + r
                j, lane = p // _L, p % _L
                pltpu.async_copy(v_hbm.at[tn[j][lane], hn[j][lane]],
                                 nb.at[r], sems[b])

        def drain(b):
            ub, vb, nb = bufs[b]
            pltpu.make_async_copy(v_hbm.at[pl.ds(0, _CH), 0], ub,
                                  sems[b]).wait()
            pltpu.make_async_copy(v_hbm.at[pl.ds(0, _CH), 0], vb,
                                  sems[b]).wait()
            pltpu.make_async_copy(v_hbm.at[pl.ds(0, _CH * _NEG), 0], nb,
                                  sems[b]).wait()

        def compute(vecs, g, b):
            iu, iv, _ = vecs
            ub, vb, nb = bufs[b]
            slot = jnp.bitwise_and(iota, _CH - 1)
            subu = jnp.bitwise_and(iu, 7)
            subv = jnp.bitwise_and(iv, 7)
            nslot = [slot * _NEG + n for n in range(_NEG)]
            half = jnp.bitwise_and(lax.shift_right_logical(iota, 3), 1)
            subn = [
                jnp.bitwise_and(
                    plsc.load_gather(
                        idxn,
                        [g * _L * _NEG + half * (_CH * _NEG)
                         + slot * _NEG + n]), 7)
                for n in range(_NEG)
            ]

            def dbody(dd, acc):
                col = jnp.full((_L,), dd, jnp.int32)
                uval = plsc.load_gather(ub, [slot, subu, col])
                vval = plsc.load_gather(vb, [slot, subv, col])
                new0 = acc[0] + uval * vval
                rest = tuple(
                    acc[1 + n]
                    + uval * plsc.load_gather(nb, [nslot[n], subn[n], col])
                    for n in range(_NEG))
                return (new0,) + rest

            z = jnp.zeros((_L,), jnp.float32)
            return lax.fori_loop(0, _D, dbody, (z,) * (1 + _NEG))

        v0 = pair_vecs(0)
        fire(v0, 0)
        fire(v0, 1)

        def pair_body(g, carry):
            vecs = pair_vecs(g)
            gn = jnp.minimum(g + 1, _NPAIR - 1)
            vecs_n = pair_vecs(gn)
            drain(0)
            acc_a = compute(vecs, g, 0)

            @pl.when(g < _NPAIR - 1)
            def _fire_a():
                fire(vecs_n, 0)

            drain(1)
            acc_b = compute(vecs, g, 1)

            @pl.when(g < _NPAIR - 1)
            def _fire_b():
                fire(vecs_n, 1)

            res = [jnp.where(lo_mask, a, bb) for a, bb in zip(acc_a, acc_b)]
            psc[pl.ds(g * _L, _L)] = res[0]
            for n in range(_NEG):
                nsc[n][pl.ds(g * _L, _L)] = res[1 + n]
            return carry

        lax.fori_loop(0, _NPAIR, pair_body, 0)

        pltpu.sync_copy(psc, pos_out.at[pl.ds(base, _CHUNK)])
        for n in range(_NEG):
            pltpu.sync_copy(nsc[n],
                            neg_out.at[pl.ds(n * _B + base, _CHUNK)])

    return k(u4, v4, pos_u, pos_v, negf)


def _loss_body(p_ref, n_ref, o_ref):
    s = jnp.clip(p_ref[...], -10.0, 10.0)
    t1 = jnp.sum(jnp.log(1.0 + jnp.exp(-s)))
    ns = jnp.clip(n_ref[...], -10.0, 10.0)
    t2 = jnp.sum(jnp.log(1.0 + jnp.exp(ns)))
    o_ref[...] = jnp.reshape((t1 + t2) * (1.0 / _B), (1, 1))


def _loss_tc(pos_sc, neg_sc):
    out = pl.pallas_call(
        _loss_body,
        out_shape=jax.ShapeDtypeStruct((1, 1), jnp.float32),
    )(pos_sc.reshape(_B // 128, 128), neg_sc.reshape(_B * _NEG // 128, 128))
    return out[0, 0]


def kernel(u_emb, v_emb, pos_u, pos_v, neg_v):
    v4 = v_emb.reshape(_V // 16, 2, 8, _D)
    negf = neg_v.reshape(_B * _NEG)
    pos_sc, neg_sc = _sc_scores(u_emb, v4, pos_u, pos_v, negf)
    return _loss_tc(pos_sc, neg_sc)


# speedup vs baseline: 1.0638x; 1.0638x over previous
import functools

import jax
import jax.numpy as jnp
from jax import lax
from jax.experimental import pallas as pl
from jax.experimental.pallas import tpu as pltpu
from jax.experimental.pallas import tpu_sc as plsc

_V = 1000000
_D = 64
_B = 16384
_NEG = 5

_NC = 2
_NS = 16
_NW = _NC * _NS
_L = 16

_CHUNK = _B // _NW
_CH = 8
_NPAIR = _CHUNK // _L


def _sc_scores(u4, v4, pos_u, pos_v, negf):
    mesh = plsc.VectorSubcoreMesh(
        core_axis_name="c", subcore_axis_name="s",
        num_cores=_NC, num_subcores=_NS)

    @functools.partial(
        pl.kernel,
        out_type=(
            jax.ShapeDtypeStruct((_B,), jnp.float32),
            jax.ShapeDtypeStruct((_NEG * _B,), jnp.float32),
        ),
        mesh=mesh,
        scratch_types=[
            pltpu.VMEM((_CHUNK,), jnp.int32),
            pltpu.VMEM((_CHUNK,), jnp.int32),
            pltpu.VMEM((_CHUNK * _NEG,), jnp.int32),
            [[pltpu.VMEM((_CH, 8, _D), jnp.float32),
              pltpu.VMEM((_CH, 8, _D), jnp.float32),
              pltpu.VMEM((_CH * _NEG, 8, _D), jnp.float32)]
             for _ in range(2)],
            pltpu.VMEM((_CHUNK,), jnp.float32),
            [pltpu.VMEM((_CHUNK,), jnp.float32) for _ in range(_NEG)],
            [pltpu.SemaphoreType.DMA for _ in range(2)],
        ],
        compiler_params=pltpu.CompilerParams(needs_layout_passes=False),
    )
    def k(u_hbm, v_hbm, pu_hbm, pv_hbm, nv_hbm, pos_out, neg_out,
          idxu, idxv, idxn, bufs, psc, nsc, sems):
        wid = lax.axis_index("s") * _NC + lax.axis_index("c")
        base = wid * _CHUNK
        pltpu.sync_copy(pu_hbm.at[pl.ds(base, _CHUNK)], idxu)
        pltpu.sync_copy(pv_hbm.at[pl.ds(base, _CHUNK)], idxv)
        pltpu.sync_copy(nv_hbm.at[pl.ds(base * _NEG, _CHUNK * _NEG)], idxn)
        iota = lax.iota(jnp.int32, _L)
        lo_mask = iota < _CH

        def pair_vecs(g):
            iu = idxu[pl.ds(g * _L, _L)]
            iv = idxv[pl.ds(g * _L, _L)]
            inj = [idxn[pl.ds(g * _L * _NEG + _L * j, _L)]
                   for j in range(_NEG)]
            return iu, iv, inj

        def fire(vecs, b):
            iu, iv, inj = vecs
            tu = lax.shift_right_logical(iu, 4)
            tv = lax.shift_right_logical(iv, 4)
            hu = jnp.bitwise_and(lax.shift_right_logical(iu, 3), 1)
            hv = jnp.bitwise_and(lax.shift_right_logical(iv, 3), 1)
            tn = [lax.shift_right_logical(x, 4) for x in inj]
            hn = [jnp.bitwise_and(lax.shift_right_logical(x, 3), 1)
                  for x in inj]
            ub, vb, nb = bufs[b]
            for t in range(_CH):
                ll = _CH * b + t
                pltpu.async_copy(u_hbm.at[tu[ll], hu[ll]], ub.at[t], sems[b])
                pltpu.async_copy(v_hbm.at[tv[ll], hv[ll]], vb.at[t], sems[b])
            for r in range(_CH * _NEG):
                p = _CH * _NEG * b + r
                j, lane = p // _L, p % _L
                pltpu.async_copy(v_hbm.at[tn[j][lane], hn[j][lane]],
                                 nb.at[r], sems[b])

        def drain(b):
            ub, vb, nb = bufs[b]
            pltpu.make_async_copy(v_hbm.at[pl.ds(0, _CH), 0], ub,
                                  sems[b]).wait()
            pltpu.make_async_copy(v_hbm.at[pl.ds(0, _CH), 0], vb,
                                  sems[b]).wait()
            pltpu.make_async_copy(v_hbm.at[pl.ds(0, _CH * _NEG), 0], nb,
                                  sems[b]).wait()

        def compute(vecs, g, b):
            iu, iv, _ = vecs
            ub, vb, nb = bufs[b]
            slot = jnp.bitwise_and(iota, _CH - 1)
            subu = jnp.bitwise_and(iu, 7)
            subv = jnp.bitwise_and(iv, 7)
            nslot = [slot * _NEG + n for n in range(_NEG)]
            half = jnp.bitwise_and(lax.shift_right_logical(iota, 3), 1)
            subn = [
                jnp.bitwise_and(
                    plsc.load_gather(
                        idxn,
                        [g * _L * _NEG + half * (_CH * _NEG)
                         + slot * _NEG + n]), 7)
                for n in range(_NEG)
            ]

            def dbody(dd, acc):
                col = jnp.full((_L,), dd, jnp.int32)
                uval = plsc.load_gather(ub, [slot, subu, col])
                vval = plsc.load_gather(vb, [slot, subv, col])
                new0 = acc[0] + uval * vval
                rest = tuple(
                    acc[1 + n]
                    + uval * plsc.load_gather(nb, [nslot[n], subn[n], col])
                    for n in range(_NEG))
                return (new0,) + rest

            z = jnp.zeros((_L,), jnp.float32)
            return lax.fori_loop(0, _D, dbody, (z,) * (1 + _NEG))

        v0 = pair_vecs(0)
        fire(v0, 0)
        fire(v0, 1)

        def pair_body(g, carry):
            vecs = pair_vecs(g)
            gn = jnp.minimum(g + 1, _NPAIR - 1)
            vecs_n = pair_vecs(gn)
            drain(0)
            acc_a = compute(vecs, g, 0)

            @pl.when(g < _NPAIR - 1)
            def _fire_a():
                fire(vecs_n, 0)

            drain(1)
            acc_b = compute(vecs, g, 1)

            @pl.when(g < _NPAIR - 1)
            def _fire_b():
                fire(vecs_n, 1)

            res = [jnp.where(lo_mask, a, bb) for a, bb in zip(acc_a, acc_b)]
            psc[pl.ds(g * _L, _L)] = res[0]
            for n in range(_NEG):
                nsc[n][pl.ds(g * _L, _L)] = res[1 + n]
            return carry

        lax.fori_loop(0, _NPAIR, pair_body, 0)

        pltpu.sync_copy(psc, pos_out.at[pl.ds(base, _CHUNK)])
        for n in range(_NEG):
            pltpu.sync_copy(nsc[n],
                            neg_out.at[pl.ds(n * _B + base, _CHUNK)])

    return k(u4, v4, pos_u, pos_v, negf)


def _loss_body(p_ref, n_ref, o_ref):
    s = jnp.clip(p_ref[...], -10.0, 10.0)
    t1 = jnp.sum(jnp.log(1.0 + jnp.exp(-s)))
    ns = jnp.clip(n_ref[...], -10.0, 10.0)
    t2 = jnp.sum(jnp.log(1.0 + jnp.exp(ns)))
    o_ref[...] = jnp.reshape((t1 + t2) * (1.0 / _B), (1, 1))


def _loss_tc(pos_sc, neg_sc):
    out = pl.pallas_call(
        _loss_body,
        out_shape=jax.ShapeDtypeStruct((1, 1), jnp.float32),
    )(pos_sc.reshape(_B // 128, 128), neg_sc.reshape(_B * _NEG // 128, 128))
    return out[0, 0]


def kernel(u_emb, v_emb, pos_u, pos_v, neg_v):
    u4 = u_emb.reshape(_V // 16, 2, 8, _D)
    v4 = v_emb.reshape(_V // 16, 2, 8, _D)
    negf = neg_v.reshape(_B * _NEG)
    pos_sc, neg_sc = _sc_scores(u4, v4, pos_u, pos_v, negf)
    return _loss_tc(pos_sc, neg_sc)
